# baseline (speedup 1.0000x reference)
import functools

import jax
import jax.numpy as jnp
from jax import lax
from jax.experimental import pallas as pl
from jax.experimental.pallas import tpu as pltpu
from jax.experimental.pallas import tpu_sc as plsc

NC = 2
NS = 16
L = 16
NW = NC * NS
CHUNK = 32
DEG_CHUNK = 128


def _sc_mesh():
    return plsc.VectorSubcoreMesh(
        core_axis_name="c", subcore_axis_name="s", num_cores=NC, num_subcores=NS
    )


def _make_deg_kernel(e_pad: int, n_pad: int):
    per_w = e_pad // NW
    n_chunks = per_w // DEG_CHUNK
    n_slice = n_pad // NS

    @functools.partial(
        pl.kernel,
        out_type=jax.ShapeDtypeStruct((NC * n_pad,), jnp.float32),
        mesh=_sc_mesh(),
        scratch_types=[
            pltpu.VMEM_SHARED((n_pad,), jnp.float32),
            pltpu.VMEM((DEG_CHUNK,), jnp.int32),
            pltpu.VMEM((DEG_CHUNK,), jnp.float32),
        ],
    )
    def deg_kernel(col_h, ew_h, z1_h, degp_h, deg_sh, cidx_v, ew_v):
        c = lax.axis_index("c")
        s = lax.axis_index("s")
        wid = c * NS + s
        pltpu.sync_copy(z1_h, deg_sh.at[pl.ds(s * n_slice, n_slice)])
        plsc.subcore_barrier()

        def chunk_body(i, carry):
            base = wid * per_w + i * DEG_CHUNK
            pltpu.sync_copy(col_h.at[pl.ds(base, DEG_CHUNK)], cidx_v)
            pltpu.sync_copy(ew_h.at[pl.ds(base, DEG_CHUNK)], ew_v)
            pltpu.sync_copy(ew_v, deg_sh.at[cidx_v], add=True)
            return carry

        lax.fori_loop(0, n_chunks, chunk_body, 0)
        plsc.subcore_barrier()
        pltpu.sync_copy(
            deg_sh.at[pl.ds(s * n_slice, n_slice)],
            degp_h.at[pl.ds(c * n_pad + s * n_slice, n_slice)],
        )

    return deg_kernel


def _make_scatter_kernel(e_pad: int, n_pk: int, n_acc: int, d: int):
    per_w = e_pad // NW
    n_chunks = per_w // CHUNK
    assert n_chunks % 2 == 0
    acc_per_tile = n_acc // NS
    pk_per_tile = n_pk // NS

    @functools.partial(
        pl.kernel,
        out_type=jax.ShapeDtypeStruct((NC, n_acc, d), jnp.float32),
        mesh=_sc_mesh(),
        scratch_types=[
            pltpu.VMEM_SHARED((n_pk, d), jnp.float32),
            pltpu.VMEM_SHARED((n_acc, d), jnp.float32),
            [pltpu.VMEM((CHUNK,), jnp.int32)] * 2,
            [pltpu.VMEM((CHUNK,), jnp.int32)] * 2,
            [pltpu.VMEM((CHUNK,), jnp.int32)] * 2,
            [pltpu.VMEM((CHUNK,), jnp.float32)] * 2,
            [pltpu.VMEM((CHUNK, d), jnp.float32)] * 2,
            [pltpu.SemaphoreType.DMA] * 2,
            [pltpu.SemaphoreType.DMA] * 2,
            [pltpu.SemaphoreType.DMA] * 2,
            [pltpu.SemaphoreType.DMA] * 2,
            [pltpu.SemaphoreType.DMA] * 2,
        ],
    )
    def scatter_kernel(rh_h, rp_h, col_h, ew_h, h2p_h, z2_h, outp_h,
                       h2_sh, acc_sh, rh, rp, cidx, ew, rows,
                       gsem, hsem, psem, csem, esem):
        c = lax.axis_index("c")
        s = lax.axis_index("s")
        wid = c * NS + s
        base0 = wid * per_w

        def scale(rows_v, rp_v, ew_v):
            for g in range(CHUNK // L):
                wvec = ew_v[pl.ds(g * L, L)]
                pvec = rp_v[pl.ds(g * L, L)]
                ws = []
                shs = []
                for t in range(L):
                    sel = jnp.full((L,), t, dtype=jnp.int32)
                    ws.append(jnp.take_along_axis(wvec, sel, axis=0))
                    pb = jnp.take_along_axis(pvec, sel, axis=0)
                    shs.append(jnp.int32(16) - (pb << 4))
                for t in range(L):
                    e = g * L + t
                    for j in range(d // L):
                        sl = pl.ds(j * L, L)
                        w = lax.bitcast_convert_type(rows_v[e, sl], jnp.int32)
                        val = lax.bitcast_convert_type(
                            (w << shs[t]) & jnp.int32(-65536), jnp.float32)
                        rows_v[e, sl] = val * ws[t]

        acc_rows = pl.ds(s * acc_per_tile, acc_per_tile)
        pk_rows = pl.ds(s * pk_per_tile, pk_per_tile)
        pltpu.sync_copy(h2p_h.at[pk_rows, :], h2_sh.at[pk_rows, :])
        pltpu.sync_copy(z2_h, acc_sh.at[acc_rows, :])
        pltpu.sync_copy(rh_h.at[pl.ds(base0, CHUNK)], rh[0])
        pltpu.sync_copy(rp_h.at[pl.ds(base0, CHUNK)], rp[0])
        pltpu.sync_copy(col_h.at[pl.ds(base0, CHUNK)], cidx[0])
        pltpu.sync_copy(ew_h.at[pl.ds(base0, CHUNK)], ew[0])
        plsc.subcore_barrier()
        pltpu.async_copy(h2_sh.at[rh[0]], rows[0], gsem[0])
        pltpu.async_copy(rh_h.at[pl.ds(base0 + CHUNK, CHUNK)], rh[1], hsem[1])

        def step(k, p):
            q = 1 - p
            pltpu.make_async_copy(h2_sh.at[rh[p]], rows[p], gsem[p]).wait()

            @pl.when(k + 2 < n_chunks)
            def _():
                pltpu.async_copy(
                    rh_h.at[pl.ds(base0 + (k + 2) * CHUNK, CHUNK)],
                    rh[p], hsem[p])

            @pl.when(k + 1 < n_chunks)
            def _():
                pltpu.make_async_copy(
                    rh_h.at[pl.ds(0, CHUNK)], rh[q], hsem[q]).wait()
                pltpu.async_copy(h2_sh.at[rh[q]], rows[q], gsem[q])
                pltpu.async_copy(
                    rp_h.at[pl.ds(base0 + (k + 1) * CHUNK, CHUNK)],
                    rp[q], psem[q])
                pltpu.async_copy(
                    col_h.at[pl.ds(base0 + (k + 1) * CHUNK, CHUNK)],
                    cidx[q], csem[q])
                pltpu.async_copy(
                    ew_h.at[pl.ds(base0 + (k + 1) * CHUNK, CHUNK)],
                    ew[q], esem[q])

            scale(rows[p], rp[p], ew[p])

            @pl.when(k + 1 < n_chunks)
            def _():
                pltpu.make_async_copy(
                    rp_h.at[pl.ds(0, CHUNK)], rp[q], psem[q]).wait()
                pltpu.make_async_copy(
                    col_h.at[pl.ds(0, CHUNK)], cidx[q], csem[q]).wait()
                pltpu.make_async_copy(
                    ew_h.at[pl.ds(0, CHUNK)], ew[q], esem[q]).wait()

            pltpu.sync_copy(rows[p], acc_sh.at[cidx[p]], add=True)

        def pair_body(i2, carry):
            step(2 * i2, 0)
            step(2 * i2 + 1, 1)
            return carry

        lax.fori_loop(0, n_chunks // 2, pair_body, 0)
        plsc.subcore_barrier()
        pltpu.sync_copy(
            acc_sh.at[acc_rows, :],
            outp_h.at[c, acc_rows, :],
        )

    return scatter_kernel


def _dense_body(x_ref, w_ref, degp_ref, b_ref, h2p_ref, selfb_ref, dis_ref):
    h = jnp.dot(x_ref[...], w_ref[...], preferred_element_type=jnp.float32)
    deg = degp_ref[0, :] + degp_ref[1, :] + 1.0
    dis = jnp.where(deg > 0, lax.rsqrt(deg), 0.0)
    h2 = h * dis[:, None]
    bn, d = h.shape
    h2r = h2.reshape(bn // 2, 2, d)
    even = h2r[:, 0, :].astype(jnp.bfloat16)
    odd = h2r[:, 1, :].astype(jnp.bfloat16)
    even_u = lax.bitcast_convert_type(even, jnp.uint16).astype(jnp.uint32)
    odd_u = lax.bitcast_convert_type(odd, jnp.uint16).astype(jnp.uint32)
    packed = even_u | (odd_u << 16)
    h2p_ref[...] = lax.bitcast_convert_type(packed, jnp.float32)
    selfb_ref[...] = h * (dis * dis)[:, None] + b_ref[...]
    dis_ref[...] = dis[:, None]


def _combine_body(p_ref, dis_ref, selfb_ref, o_ref):
    o_ref[...] = (p_ref[0] + p_ref[1]) * dis_ref[...] + selfb_ref[...]


def kernel(x, edge_index, edge_attr, W, b):
    n, d_in = x.shape
    d_out = W.shape[1]
    row = edge_index[0].astype(jnp.int32)
    col = edge_index[1].astype(jnp.int32)
    ew = edge_attr.astype(jnp.float32)

    e = row.shape[0]
    e_pad = -(-e // (NW * CHUNK * 2)) * (NW * CHUNK * 2)
    pad = e_pad - e
    if pad:
        row = jnp.concatenate([row, jnp.zeros((pad,), jnp.int32)])
        col = jnp.concatenate([col, jnp.zeros((pad,), jnp.int32)])
        ew = jnp.concatenate([ew, jnp.zeros((pad,), jnp.float32)])
    rh = row >> 1
    rp = row & 1

    n_pad = -(-n // (NS * 128)) * (NS * 128)
    z1 = jnp.zeros((n_pad // NS,), jnp.float32)
    degp = _make_deg_kernel(e_pad, n_pad)(col, ew, z1).reshape(NC, n_pad)

    bn = 512
    n2 = -(-n // bn) * bn
    x_p = jnp.pad(x, ((0, n2 - n), (0, 0))) if n2 != n else x
    degp2 = (jnp.pad(degp, ((0, 0), (0, n2 - n_pad))) if n2 > n_pad
             else degp[:, :n2])
    grid = n2 // bn
    h2p, selfb, dis = pl.pallas_call(
        _dense_body,
        grid=(grid,),
        in_specs=[
            pl.BlockSpec((bn, d_in), lambda i: (i, 0)),
            pl.BlockSpec((d_in, d_out), lambda i: (0, 0)),
            pl.BlockSpec((NC, bn), lambda i: (0, i)),
            pl.BlockSpec((1, d_out), lambda i: (0, 0)),
        ],
        out_specs=[
            pl.BlockSpec((bn // 2, d_out), lambda i: (i, 0)),
            pl.BlockSpec((bn, d_out), lambda i: (i, 0)),
            pl.BlockSpec((bn, 1), lambda i: (i, 0)),
        ],
        out_shape=[
            jax.ShapeDtypeStruct((n2 // 2, d_out), jnp.float32),
            jax.ShapeDtypeStruct((n2, d_out), jnp.float32),
            jax.ShapeDtypeStruct((n2, 1), jnp.float32),
        ],
    )(x_p, W, degp2, b.reshape(1, d_out))

    n_acc = -(-n // (NS * 8)) * (NS * 8)
    z2 = jnp.zeros((n_acc // NS, d_out), jnp.float32)
    partial = _make_scatter_kernel(e_pad, n2 // 2, n_acc, d_out)(
        rh, rp, col, ew, h2p, z2)

    bn2 = 1000
    grid2 = n // bn2
    out = pl.pallas_call(
        _combine_body,
        grid=(grid2,),
        in_specs=[
            pl.BlockSpec((NC, bn2, d_out), lambda i: (0, i, 0)),
            pl.BlockSpec((bn2, 1), lambda i: (i, 0)),
            pl.BlockSpec((bn2, d_out), lambda i: (i, 0)),
        ],
        out_specs=pl.BlockSpec((bn2, d_out), lambda i: (i, 0)),
        out_shape=jax.ShapeDtypeStruct((n, d_out), jnp.float32),
    )(partial, dis, selfb)
    return out
